# baseline (device time: 136472 ns/iter reference)
import jax
import jax.numpy as jnp
from jax import lax
from jax.experimental import pallas as pl
from jax.experimental.pallas import tpu as pltpu

C = 16


def kernel(x):
    m_per, n = x.shape
    half = m_per // 2
    rows = half // C

    def body(x_ref, out_ref, local_sem, xs_sems, xr_sems, ys_sems, yr_sems,
             zs_sems, zr_sems):
        my_x = lax.axis_index("x")
        my_y = lax.axis_index("y")
        my_z = lax.axis_index("z")
        xpeer = (1 - my_x, my_y, my_z)
        ypeer = (my_x, 1 - my_y, my_z)
        zpeer = (my_x, my_y, my_z + 1 - 2 * (my_z % 2))

        barrier_sem = pltpu.get_barrier_semaphore()
        for nbr in (xpeer, ypeer, zpeer):
            pl.semaphore_signal(
                barrier_sem, inc=1, device_id=nbr,
                device_id_type=pl.DeviceIdType.MESH,
            )
        pl.semaphore_wait(barrier_sem, 3)

        my_base = my_x * m_per
        fo_base = (1 - my_x) * m_per

        x_rdmas = []
        for c in range(C):
            off = my_y * half + c * rows
            rdma = pltpu.make_async_remote_copy(
                src_ref=x_ref.at[pl.ds(off, rows), :],
                dst_ref=out_ref.at[pl.ds(my_base + off, rows), :],
                send_sem=xs_sems.at[c],
                recv_sem=xr_sems.at[c],
                device_id=xpeer,
                device_id_type=pl.DeviceIdType.MESH,
            )
            rdma.start()
            x_rdmas.append(rdma)


        y_rdmas = []
        for c in range(C):
            off = my_y * half + c * rows
            rdma = pltpu.make_async_remote_copy(
                src_ref=x_ref.at[pl.ds(off, rows), :],
                dst_ref=out_ref.at[pl.ds(fo_base + off, rows), :],
                send_sem=ys_sems.at[c],
                recv_sem=yr_sems.at[c],
                device_id=ypeer,
                device_id_type=pl.DeviceIdType.MESH,
            )
            rdma.start()
            y_rdmas.append(rdma)
        z_rdmas = []
        for c in range(C):
            off = my_y * half + c * rows
            rdma = pltpu.make_async_remote_copy(
                src_ref=x_ref.at[pl.ds(off, rows), :],
                dst_ref=out_ref.at[pl.ds(my_base + off, rows), :],
                send_sem=zs_sems.at[c],
                recv_sem=zr_sems.at[c],
                device_id=zpeer,
                device_id_type=pl.DeviceIdType.MESH,
            )
            rdma.start()
            z_rdmas.append(rdma)

        for c in range(C):
            x_rdmas[c].wait_recv()

        for c in range(C):
            x_rdmas[c].wait_send()
            y_rdmas[c].wait_send()
            y_rdmas[c].wait_recv()
            z_rdmas[c].wait_send()
            z_rdmas[c].wait_recv()

    return pl.pallas_call(
        body,
        out_shape=jax.ShapeDtypeStruct((2 * m_per, n), x.dtype),
        in_specs=[pl.BlockSpec(memory_space=pltpu.VMEM)],
        out_specs=pl.BlockSpec(memory_space=pltpu.VMEM),
        scratch_shapes=[
            pltpu.SemaphoreType.DMA,
            pltpu.SemaphoreType.DMA((C,)),
            pltpu.SemaphoreType.DMA((C,)),
            pltpu.SemaphoreType.DMA((C,)),
            pltpu.SemaphoreType.DMA((C,)),
            pltpu.SemaphoreType.DMA((C,)),
            pltpu.SemaphoreType.DMA((C,)),
        ],
        compiler_params=pltpu.CompilerParams(collective_id=0),
    )(x)


# device time: 135172 ns/iter; 1.0096x vs baseline; 1.0096x over previous
import jax
import jax.numpy as jnp
from jax import lax
from jax.experimental import pallas as pl
from jax.experimental.pallas import tpu as pltpu

C = 4


def kernel(x):
    m_per, n = x.shape
    half = m_per // 2
    rows = half // C

    def body(x_ref, out_ref, local_sem, xs_sems, xr_sems, ys_sems, yr_sems,
             zs_sems, zr_sems):
        my_x = lax.axis_index("x")
        my_y = lax.axis_index("y")
        my_z = lax.axis_index("z")
        xpeer = (1 - my_x, my_y, my_z)
        ypeer = (my_x, 1 - my_y, my_z)
        zpeer = (my_x, my_y, my_z + 1 - 2 * (my_z % 2))

        barrier_sem = pltpu.get_barrier_semaphore()
        for nbr in (xpeer, ypeer, zpeer):
            pl.semaphore_signal(
                barrier_sem, inc=1, device_id=nbr,
                device_id_type=pl.DeviceIdType.MESH,
            )
        pl.semaphore_wait(barrier_sem, 3)

        my_base = my_x * m_per
        fo_base = (1 - my_x) * m_per

        x_rdmas = []
        for c in range(C):
            off = my_y * half + c * rows
            rdma = pltpu.make_async_remote_copy(
                src_ref=x_ref.at[pl.ds(off, rows), :],
                dst_ref=out_ref.at[pl.ds(my_base + off, rows), :],
                send_sem=xs_sems.at[c],
                recv_sem=xr_sems.at[c],
                device_id=xpeer,
                device_id_type=pl.DeviceIdType.MESH,
            )
            rdma.start()
            x_rdmas.append(rdma)


        y_rdmas = []
        for c in range(C):
            off = my_y * half + c * rows
            rdma = pltpu.make_async_remote_copy(
                src_ref=x_ref.at[pl.ds(off, rows), :],
                dst_ref=out_ref.at[pl.ds(fo_base + off, rows), :],
                send_sem=ys_sems.at[c],
                recv_sem=yr_sems.at[c],
                device_id=ypeer,
                device_id_type=pl.DeviceIdType.MESH,
            )
            rdma.start()
            y_rdmas.append(rdma)
        z_rdmas = []
        for c in range(C):
            off = my_y * half + c * rows
            rdma = pltpu.make_async_remote_copy(
                src_ref=x_ref.at[pl.ds(off, rows), :],
                dst_ref=out_ref.at[pl.ds(my_base + off, rows), :],
                send_sem=zs_sems.at[c],
                recv_sem=zr_sems.at[c],
                device_id=zpeer,
                device_id_type=pl.DeviceIdType.MESH,
            )
            rdma.start()
            z_rdmas.append(rdma)

        for c in range(C):
            x_rdmas[c].wait_recv()

        for c in range(C):
            x_rdmas[c].wait_send()
            y_rdmas[c].wait_send()
            y_rdmas[c].wait_recv()
            z_rdmas[c].wait_send()
            z_rdmas[c].wait_recv()

    return pl.pallas_call(
        body,
        out_shape=jax.ShapeDtypeStruct((2 * m_per, n), x.dtype),
        in_specs=[pl.BlockSpec(memory_space=pltpu.VMEM)],
        out_specs=pl.BlockSpec(memory_space=pltpu.VMEM),
        scratch_shapes=[
            pltpu.SemaphoreType.DMA,
            pltpu.SemaphoreType.DMA((C,)),
            pltpu.SemaphoreType.DMA((C,)),
            pltpu.SemaphoreType.DMA((C,)),
            pltpu.SemaphoreType.DMA((C,)),
            pltpu.SemaphoreType.DMA((C,)),
            pltpu.SemaphoreType.DMA((C,)),
        ],
        compiler_params=pltpu.CompilerParams(collective_id=0),
    )(x)


# device time: 131836 ns/iter; 1.0352x vs baseline; 1.0253x over previous
import jax
import jax.numpy as jnp
from jax import lax
from jax.experimental import pallas as pl
from jax.experimental.pallas import tpu as pltpu

C = 16


def kernel(x):
    m_per, n = x.shape
    half = m_per // 2
    rows = half // C

    def body(x_ref, out_ref, local_sem, xs_sems, xr_sems, ys_sems, yr_sems):
        my_x = lax.axis_index("x")
        my_y = lax.axis_index("y")
        my_z = lax.axis_index("z")
        xpeer = (1 - my_x, my_y, my_z)
        ypeer = (my_x, 1 - my_y, my_z)

        barrier_sem = pltpu.get_barrier_semaphore()
        for nbr in (xpeer, ypeer):
            pl.semaphore_signal(
                barrier_sem, inc=1, device_id=nbr,
                device_id_type=pl.DeviceIdType.MESH,
            )
        pl.semaphore_wait(barrier_sem, 2)

        my_base = my_x * m_per
        fo_base = (1 - my_x) * m_per

        x_rdmas = []
        for c in range(C):
            off = my_y * half + c * rows
            rdma = pltpu.make_async_remote_copy(
                src_ref=x_ref.at[pl.ds(off, rows), :],
                dst_ref=out_ref.at[pl.ds(my_base + off, rows), :],
                send_sem=xs_sems.at[c],
                recv_sem=xr_sems.at[c],
                device_id=xpeer,
                device_id_type=pl.DeviceIdType.MESH,
            )
            rdma.start()
            x_rdmas.append(rdma)

        local = pltpu.make_async_copy(
            x_ref, out_ref.at[pl.ds(my_base, m_per), :], local_sem
        )
        local.start()

        y_rdmas = []
        for c in range(C):
            x_rdmas[c].wait_recv()
            off = fo_base + my_y * half + c * rows
            rdma = pltpu.make_async_remote_copy(
                src_ref=out_ref.at[pl.ds(off, rows), :],
                dst_ref=out_ref.at[pl.ds(off, rows), :],
                send_sem=ys_sems.at[c],
                recv_sem=yr_sems.at[c],
                device_id=ypeer,
                device_id_type=pl.DeviceIdType.MESH,
            )
            rdma.start()
            y_rdmas.append(rdma)

        local.wait()
        for c in range(C):
            x_rdmas[c].wait_send()
            y_rdmas[c].wait_send()
            y_rdmas[c].wait_recv()

    return pl.pallas_call(
        body,
        out_shape=jax.ShapeDtypeStruct((2 * m_per, n), x.dtype),
        in_specs=[pl.BlockSpec(memory_space=pl.ANY)],
        out_specs=pl.BlockSpec(memory_space=pl.ANY),
        scratch_shapes=[
            pltpu.SemaphoreType.DMA,
            pltpu.SemaphoreType.DMA((C,)),
            pltpu.SemaphoreType.DMA((C,)),
            pltpu.SemaphoreType.DMA((C,)),
            pltpu.SemaphoreType.DMA((C,)),
        ],
        compiler_params=pltpu.CompilerParams(collective_id=0),
    )(x)


# device time: 110754 ns/iter; 1.2322x vs baseline; 1.1903x over previous
import jax
import jax.numpy as jnp
from jax import lax
from jax.experimental import pallas as pl
from jax.experimental.pallas import tpu as pltpu

K = 8


def kernel(x):
    m_per, n = x.shape
    qrows = m_per // 4
    rows = qrows // K
    H = K // 2

    def body(x_ref, out_ref, local_sem,
             xs, xr, yps, ypr, zps, zpr, yss, ysr, zss, zsr):
        my_x = lax.axis_index("x")
        my_y = lax.axis_index("y")
        my_z = lax.axis_index("z")
        s = my_z % 2
        xpeer = (1 - my_x, my_y, my_z)
        ypeer = (my_x, 1 - my_y, my_z)
        zpeer = (my_x, my_y, my_z + 1 - 2 * s)

        g_own = 2 * my_y + s
        g_yp = 2 * (1 - my_y) + s
        g_zp = 2 * my_y + (1 - s)

        barrier_sem = pltpu.get_barrier_semaphore()
        for nbr in (xpeer, ypeer, zpeer):
            pl.semaphore_signal(
                barrier_sem, inc=1, device_id=nbr,
                device_id_type=pl.DeviceIdType.MESH,
            )
        pl.semaphore_wait(barrier_sem, 3)

        my_base = my_x * m_per
        fo_base = (1 - my_x) * m_per

        def remote(off, send_sem, recv_sem, dev):
            return pltpu.make_async_remote_copy(
                src_ref=out_ref.at[pl.ds(off, rows), :],
                dst_ref=out_ref.at[pl.ds(off, rows), :],
                send_sem=send_sem,
                recv_sem=recv_sem,
                device_id=dev,
                device_id_type=pl.DeviceIdType.MESH,
            )

        x_rdmas = []
        for k in range(K):
            off = g_own * qrows + k * rows
            rdma = pltpu.make_async_remote_copy(
                src_ref=x_ref.at[pl.ds(off, rows), :],
                dst_ref=out_ref.at[pl.ds(my_base + off, rows), :],
                send_sem=xs.at[k],
                recv_sem=xr.at[k],
                device_id=xpeer,
                device_id_type=pl.DeviceIdType.MESH,
            )
            rdma.start()
            x_rdmas.append(rdma)

        local = pltpu.make_async_copy(
            x_ref, out_ref.at[pl.ds(my_base, m_per), :], local_sem
        )
        local.start()

        yp_rdmas, zp_rdmas = [], []
        for k in range(K):
            x_rdmas[k].wait_recv()
            off = fo_base + g_own * qrows + k * rows
            rdma = remote(off, yps.at[k], ypr.at[k], ypeer)
            rdma.start()
            yp_rdmas.append(rdma)
            rdma = remote(off, zps.at[k], zpr.at[k], zpeer)
            rdma.start()
            zp_rdmas.append(rdma)

        ys_rdmas, zs_rdmas = [], []
        for k in range(K):
            zp_rdmas[k].wait_recv()
            if k < H:
                off = fo_base + g_zp * qrows + k * rows
                rdma = remote(off, yss.at[k], ysr.at[k], ypeer)
                rdma.start()
                ys_rdmas.append(rdma)
            yp_rdmas[k].wait_recv()
            if k >= H:
                off = fo_base + g_yp * qrows + k * rows
                rdma = remote(off, zss.at[k - H], zsr.at[k - H], zpeer)
                rdma.start()
                zs_rdmas.append(rdma)

        local.wait()
        for k in range(K):
            x_rdmas[k].wait_send()
            yp_rdmas[k].wait_send()
            zp_rdmas[k].wait_send()
        for k in range(H):
            ys_rdmas[k].wait_send()
            ys_rdmas[k].wait_recv()
            zs_rdmas[k].wait_send()
            zs_rdmas[k].wait_recv()

    return pl.pallas_call(
        body,
        out_shape=jax.ShapeDtypeStruct((2 * m_per, n), x.dtype),
        in_specs=[pl.BlockSpec(memory_space=pl.ANY)],
        out_specs=pl.BlockSpec(memory_space=pl.ANY),
        scratch_shapes=[
            pltpu.SemaphoreType.DMA,
            pltpu.SemaphoreType.DMA((K,)),
            pltpu.SemaphoreType.DMA((K,)),
            pltpu.SemaphoreType.DMA((K,)),
            pltpu.SemaphoreType.DMA((K,)),
            pltpu.SemaphoreType.DMA((K,)),
            pltpu.SemaphoreType.DMA((K,)),
            pltpu.SemaphoreType.DMA((H,)),
            pltpu.SemaphoreType.DMA((H,)),
            pltpu.SemaphoreType.DMA((H,)),
            pltpu.SemaphoreType.DMA((H,)),
        ],
        compiler_params=pltpu.CompilerParams(collective_id=0),
    )(x)
